# baseline (speedup 1.0000x reference)
import functools

import jax
import jax.numpy as jnp
from jax import lax
from jax.experimental import pallas as pl
from jax.experimental.pallas import tpu as pltpu
from jax.experimental.pallas import tpu_sc as plsc

TOP_K = 2
TILE_R = 256
TILE_T = 256


def _shared_routing_body(x_ref, gwt_ref, wg_ref, wu_ref, wd_ref,
                         sh_ref, w_ref, idx_ref):
    x = x_ref[...]
    xb = x.astype(jnp.bfloat16)
    g = jnp.dot(xb, wg_ref[...], preferred_element_type=jnp.float32)
    u = jnp.dot(xb, wu_ref[...], preferred_element_type=jnp.float32)
    h = (g * jax.nn.sigmoid(g) * u).astype(jnp.bfloat16)
    sh_ref[...] = jnp.dot(h, wd_ref[...], preferred_element_type=jnp.float32)

    logits = jnp.dot(x, gwt_ref[...], preferred_element_type=jnp.float32)
    m = jnp.max(logits, axis=-1, keepdims=True)
    e = jnp.exp(logits - m)
    p = e / jnp.sum(e, axis=-1, keepdims=True)
    E = p.shape[-1]
    iota = lax.broadcasted_iota(jnp.int32, p.shape, 1)
    m1 = jnp.max(p, axis=-1, keepdims=True)
    i1 = jnp.min(jnp.where(p == m1, iota, E), axis=-1, keepdims=True)
    p2 = jnp.where(iota == i1, -1.0, p)
    m2 = jnp.max(p2, axis=-1, keepdims=True)
    i2 = jnp.min(jnp.where(p2 == m2, iota, E), axis=-1, keepdims=True)
    denom = m1 + m2 + 1e-20
    col = lax.broadcasted_iota(jnp.int32, (p.shape[0], 128), 1)
    w_ref[...] = jnp.where(col == 0, m1 / denom,
                           jnp.where(col == 1, m2 / denom, 0.0))
    idx_ref[...] = jnp.where(col == 0, i1, jnp.where(col == 1, i2, 0))


def _shared_routing(x2, gate_wt, wg_s, wu_s, wd_s):
    T, D = x2.shape
    E = gate_wt.shape[1]
    MS = wg_s.shape[1]
    nt = T // TILE_T
    const = lambda i: (0, 0)
    return pl.pallas_call(
        _shared_routing_body,
        grid=(nt,),
        in_specs=[
            pl.BlockSpec((TILE_T, D), lambda i: (i, 0)),
            pl.BlockSpec((D, E), const),
            pl.BlockSpec((D, MS), const),
            pl.BlockSpec((D, MS), const),
            pl.BlockSpec((MS, D), const),
        ],
        out_specs=[
            pl.BlockSpec((TILE_T, D), lambda i: (i, 0)),
            pl.BlockSpec((TILE_T, 128), lambda i: (i, 0)),
            pl.BlockSpec((TILE_T, 128), lambda i: (i, 0)),
        ],
        out_shape=[
            jax.ShapeDtypeStruct((T, D), jnp.float32),
            jax.ShapeDtypeStruct((T, 128), jnp.float32),
            jax.ShapeDtypeStruct((T, 128), jnp.int32),
        ],
    )(x2, gate_wt, wg_s, wu_s, wd_s)


@functools.partial(jax.jit, static_argnames=("chunk",))
def _sc_gather(table, idx, chunk=64):
    V, D = table.shape
    B = idx.shape[0]
    info = plsc.get_sparse_core_info()
    nw = info.num_cores * info.num_subcores
    b_per_w = B // nw
    n_chunks = b_per_w // chunk
    mesh = plsc.VectorSubcoreMesh(core_axis_name="c", subcore_axis_name="s")

    @functools.partial(
        pl.kernel, mesh=mesh,
        out_type=jax.ShapeDtypeStruct((B, D), jnp.float32),
        scratch_types=[
            pltpu.VMEM((chunk,), jnp.int32),
            pltpu.VMEM((chunk, D), jnp.float32),
            pltpu.SemaphoreType.DMA,
        ],
    )
    def k(table_hbm, idx_hbm, out_hbm, idx_v, rows_v, sem):
        wid = lax.axis_index("s") * info.num_cores + lax.axis_index("c")
        base = wid * b_per_w
        for c in range(n_chunks):
            off = base + c * chunk
            pltpu.sync_copy(idx_hbm.at[pl.ds(off, chunk)], idx_v)
            pltpu.async_copy(table_hbm.at[idx_v], rows_v, sem).wait()
            pltpu.sync_copy(rows_v, out_hbm.at[pl.ds(off, chunk)])

    return k(table, idx)


def _expert_body(te_ref, tv_ref, xs_ref, wg_ref, wu_ref, wd_ref, ys_ref):
    i = pl.program_id(0)

    @pl.when(tv_ref[i] == 1)
    def _():
        xb = xs_ref[...].astype(jnp.bfloat16)
        g = jnp.dot(xb, wg_ref[0], preferred_element_type=jnp.float32)
        u = jnp.dot(xb, wu_ref[0], preferred_element_type=jnp.float32)
        h = (g * jax.nn.sigmoid(g) * u).astype(jnp.bfloat16)
        ys_ref[...] = jnp.dot(h, wd_ref[0], preferred_element_type=jnp.float32)


def _experts(te, tv, xs, wg, wu, wd):
    P, D = xs.shape
    M = wg.shape[2]
    nt = P // TILE_R
    grid_spec = pltpu.PrefetchScalarGridSpec(
        num_scalar_prefetch=2,
        grid=(nt,),
        in_specs=[
            pl.BlockSpec((TILE_R, D), lambda i, te, tv: (i, 0)),
            pl.BlockSpec((1, D, M), lambda i, te, tv: (te[i], 0, 0)),
            pl.BlockSpec((1, D, M), lambda i, te, tv: (te[i], 0, 0)),
            pl.BlockSpec((1, M, D), lambda i, te, tv: (te[i], 0, 0)),
        ],
        out_specs=pl.BlockSpec((TILE_R, D), lambda i, te, tv: (i, 0)),
    )
    return pl.pallas_call(
        _expert_body,
        grid_spec=grid_spec,
        out_shape=jax.ShapeDtypeStruct((P, D), jnp.float32),
    )(te, tv, xs, wg, wu, wd)


def _combine_body(sh_ref, y0_ref, y1_ref, w_ref, out_ref):
    w0 = w_ref[:, 0:1]
    w1 = w_ref[:, 1:2]
    out_ref[...] = sh_ref[...] + w0 * y0_ref[...] + w1 * y1_ref[...]


def _combine(sh, ysg, w):
    T, D = sh.shape
    nt = T // TILE_T
    return pl.pallas_call(
        _combine_body,
        grid=(nt,),
        in_specs=[
            pl.BlockSpec((TILE_T, D), lambda i: (i, 0)),
            pl.BlockSpec((TILE_T, D), lambda i: (i, 0)),
            pl.BlockSpec((TILE_T, D), lambda i: (i + nt, 0)),
            pl.BlockSpec((TILE_T, 128), lambda i: (i, 0)),
        ],
        out_specs=pl.BlockSpec((TILE_T, D), lambda i: (i, 0)),
        out_shape=jax.ShapeDtypeStruct((T, D), jnp.float32),
    )(sh, ysg, ysg, w)


def _plan(topk_idx, T, E, P_pad):
    flat_e = topk_idx.reshape(-1)
    oh = (flat_e[:, None] == jnp.arange(E)[None, :]).astype(jnp.int32)
    ranks = jnp.cumsum(oh, axis=0) - oh
    rank = jnp.take_along_axis(ranks, flat_e[:, None], axis=1)[:, 0]
    counts = jnp.sum(oh, axis=0)
    padded = ((counts + TILE_R - 1) // TILE_R) * TILE_R
    poff = jnp.concatenate([jnp.zeros((1,), jnp.int32),
                            jnp.cumsum(padded).astype(jnp.int32)])
    slot = poff[flat_e] + rank
    src_tok = jnp.zeros((P_pad,), jnp.int32).at[slot].set(
        jnp.arange(T * TOP_K, dtype=jnp.int32) // TOP_K)
    starts = jnp.arange(P_pad // TILE_R, dtype=jnp.int32) * TILE_R
    e_raw = jnp.searchsorted(poff, starts, side="right").astype(jnp.int32) - 1
    tv = (starts < poff[E]).astype(jnp.int32)
    te = jnp.where(tv == 1, jnp.clip(e_raw, 0, E - 1), 0)
    sl2 = slot.reshape(T, TOP_K)
    pos_flat = jnp.concatenate([sl2[:, 0], sl2[:, 1]])
    return src_tok, te, tv, pos_flat


def kernel(hidden_states, gate_w, Wg, Wu, Wd, Wg_s, Wu_s, Wd_s):
    b, s, d = hidden_states.shape
    T = b * s
    E = gate_w.shape[0]
    x2 = hidden_states.reshape(T, d)
    P_pad = T * TOP_K + E * TILE_R

    sh, w_pad, idx_pad = _shared_routing(
        x2, gate_w.T,
        Wg_s.astype(jnp.bfloat16), Wu_s.astype(jnp.bfloat16),
        Wd_s.astype(jnp.bfloat16))
    topk_idx = idx_pad[:, :TOP_K]

    src_tok, te, tv, pos_flat = _plan(topk_idx, T, E, P_pad)

    xs = _sc_gather(x2, src_tok)
    ys = _experts(te, tv, xs,
                  Wg.astype(jnp.bfloat16), Wu.astype(jnp.bfloat16),
                  Wd.astype(jnp.bfloat16))
    ysg = _sc_gather(ys, pos_flat)
    out = _combine(sh, ysg, w_pad)
    return out.reshape(b, s, d)
